# baseline (speedup 1.0000x reference)
import functools

import jax
import jax.numpy as jnp
from jax import lax
from jax.experimental import pallas as pl
from jax.experimental.pallas import tpu as pltpu
from jax.experimental.pallas import tpu_sc as plsc

N_NODES = 10000
N_EDGES = 320000
D_FEAT = 128
D_EDGE = 16

NUM_CORES = 2
NUM_SUBCORES = 16
NUM_TILES = NUM_CORES * NUM_SUBCORES

BLOCK_E = 64
N_BLOCKS = 5120
E_PAD = N_BLOCKS * BLOCK_E
N_PAD = 10112
ROWS_PER_SUBCORE = N_PAD // NUM_SUBCORES

EDGE_MLP_BLOCK = 8192


def _sc_aggregate(hp, srcp, dstp, em, z_ah):
    mesh = plsc.VectorSubcoreMesh(core_axis_name="c", subcore_axis_name="s")

    @functools.partial(
        pl.kernel,
        out_type=jax.ShapeDtypeStruct((NUM_CORES * N_PAD, D_FEAT),
                                      jnp.float32),
        mesh=mesh,
        scratch_types=[
            pltpu.VMEM((BLOCK_E, D_FEAT), jnp.float32),
            pltpu.VMEM_SHARED((N_PAD, D_FEAT), jnp.float32),
        ],
    )
    def sc_kernel(hp_hbm, src_hbm, dst_hbm, em_hbm, zah_hbm,
                  ah_out, rows_v, sh_ah):
        c = lax.axis_index("c")
        s = lax.axis_index("s")
        r0 = s * ROWS_PER_SUBCORE

        pltpu.sync_copy(zah_hbm, sh_ah.at[pl.ds(r0, ROWS_PER_SUBCORE)])
        plsc.subcore_barrier()

        def body(src_win, dst_win, em_win):
            pltpu.sync_copy(hp_hbm.at[src_win.at[0]], rows_v)
            pltpu.sync_copy(rows_v, sh_ah.at[dst_win.at[0]], add=True)
            pltpu.sync_copy(em_win, sh_ah.at[dst_win.at[0]], add=True)

        pltpu.emit_pipeline(
            body,
            grid=(N_BLOCKS,),
            in_specs=[
                pl.BlockSpec((1, BLOCK_E), index_map=lambda i: (i, 0)),
                pl.BlockSpec((1, BLOCK_E), index_map=lambda i: (i, 0)),
                pl.BlockSpec((BLOCK_E, D_FEAT), index_map=lambda i: (i, 0)),
            ],
            out_specs=[],
            core_axis_name=("c", "s"),
            dimension_semantics=(pltpu.PARALLEL,),
        )(src_hbm, dst_hbm, em_hbm)

        plsc.subcore_barrier()

        pltpu.sync_copy(sh_ah.at[pl.ds(r0, ROWS_PER_SUBCORE)],
                        ah_out.at[pl.ds(c * N_PAD + r0, ROWS_PER_SUBCORE)])

    return sc_kernel(hp, srcp, dstp, em, z_ah)


def _edge_mlp_body(ea_ref, we_ref, out_ref):
    out_ref[...] = jnp.dot(ea_ref[...], we_ref[...],
                           preferred_element_type=jnp.float32)


def _tc_edge_mlp(eap, We):
    return pl.pallas_call(
        _edge_mlp_body,
        grid=(E_PAD // EDGE_MLP_BLOCK,),
        in_specs=[
            pl.BlockSpec((EDGE_MLP_BLOCK, D_EDGE), lambda i: (i, 0)),
            pl.BlockSpec((D_EDGE, D_FEAT), lambda i: (0, 0)),
        ],
        out_specs=pl.BlockSpec((EDGE_MLP_BLOCK, D_FEAT), lambda i: (i, 0)),
        out_shape=jax.ShapeDtypeStruct((E_PAD, D_FEAT), jnp.float32),
    )(eap, We)


ROWS_PER_TC_BLOCK = 1000


def _tc_combine_body(ah_ref, w1_ref, b1_ref, w2_ref, b2_ref, out_ref):
    agg = ah_ref[0] + ah_ref[1]
    h1 = jnp.maximum(
        jnp.dot(agg, w1_ref[...], preferred_element_type=jnp.float32)
        + b1_ref[...], 0.0)
    out_ref[...] = (jnp.dot(h1, w2_ref[...], preferred_element_type=jnp.float32)
                    + b2_ref[...])


def _tc_combine(ah, W1, b12, W2, b22):
    grid = N_NODES // ROWS_PER_TC_BLOCK
    return pl.pallas_call(
        _tc_combine_body,
        grid=(grid,),
        in_specs=[
            pl.BlockSpec((NUM_CORES, ROWS_PER_TC_BLOCK, D_FEAT),
                         lambda i: (0, i, 0)),
            pl.BlockSpec((D_FEAT, D_FEAT), lambda i: (0, 0)),
            pl.BlockSpec((1, D_FEAT), lambda i: (0, 0)),
            pl.BlockSpec((D_FEAT, D_FEAT), lambda i: (0, 0)),
            pl.BlockSpec((1, D_FEAT), lambda i: (0, 0)),
        ],
        out_specs=pl.BlockSpec((ROWS_PER_TC_BLOCK, D_FEAT), lambda i: (i, 0)),
        out_shape=jax.ShapeDtypeStruct((N_NODES, D_FEAT), jnp.float32),
    )(ah, W1, b12, W2, b22)


def kernel(H, edge_index, edge_attr, We, be, W1, b1, W2, b2):
    src = edge_index[0].astype(jnp.int32)
    dst = edge_index[1].astype(jnp.int32)
    pad = E_PAD - N_EDGES
    srcp = jnp.concatenate([src, jnp.full((pad,), N_NODES, jnp.int32)]
                           ).reshape(N_BLOCKS, BLOCK_E)
    dstp = jnp.concatenate([dst, jnp.full((pad,), N_NODES, jnp.int32)]
                           ).reshape(N_BLOCKS, BLOCK_E)
    eap = jnp.concatenate(
        [edge_attr.astype(jnp.float32), jnp.zeros((pad, D_EDGE), jnp.float32)])
    hp = jnp.concatenate(
        [H.astype(jnp.float32) + be[None, :].astype(jnp.float32),
         jnp.zeros((N_PAD - N_NODES, D_FEAT), jnp.float32)])
    z_ah = jnp.zeros((ROWS_PER_SUBCORE, D_FEAT), jnp.float32)

    em = _tc_edge_mlp(eap, We.astype(jnp.float32))
    ah_flat = _sc_aggregate(hp, srcp, dstp, em, z_ah)
    ah = ah_flat.reshape(NUM_CORES, N_PAD, D_FEAT)

    return _tc_combine(ah,
                       W1.astype(jnp.float32), b1.reshape(1, D_FEAT),
                       W2.astype(jnp.float32), b2.reshape(1, D_FEAT))
